# Initial kernel scaffold; baseline (speedup 1.0000x reference)
import jax
import jax.numpy as jnp
from jax.experimental import pallas as pl


def kernel(x, edge_src, edge_dst, rel_ids, rel_emb_0, Wm_0, bm_0, Ws_0, bs_0, Wn_0, bn_0, rel_emb_1, Wm_1, bm_1, Ws_1, bs_1, Wn_1, bn_1):
    raise NotImplementedError("write your pallas kernel here")


# speedup vs baseline: 2.9312x; 2.9312x over previous
import functools

import jax
import jax.numpy as jnp
from jax import lax
from jax.experimental import pallas as pl
from jax.experimental.pallas import tpu as pltpu
from jax.experimental.pallas import tpu_sc as plsc

N = 10000
D = 128
R = 16
RD = 16
E = 320000

NC = 2
NS = 16
NW = NC * NS
CH = 128
CW = -(-E // (NW * CH))
EW = CW * CH
EP = EW * NW
NP = -(-N // 128) * 128
RPT = NP // NS

_HI = jax.lax.Precision.HIGHEST


def _sc_body(tbl_hbm, gidx, sidx, acc_out, idx_v, rows_v, acc_sh, sem):
    cid = lax.axis_index("c")
    sid = lax.axis_index("s")
    wid = sid * NC + cid
    zero16 = jnp.zeros((16,), jnp.float32)

    def _zrow(i, c):
        for j in range(D // 16):
            rows_v[i, pl.ds(j * 16, 16)] = zero16
        return c

    lax.fori_loop(0, CH, _zrow, 0)
    r0 = sid * RPT
    for k in range(RPT // CH):
        pltpu.sync_copy(rows_v, acc_sh.at[pl.ds(r0 + k * CH, CH)])
    rem = RPT % CH
    if rem:
        tail = r0 + (RPT // CH) * CH
        pltpu.sync_copy(rows_v.at[pl.ds(0, rem)], acc_sh.at[pl.ds(tail, rem)])
    plsc.subcore_barrier()

    ebase = wid * EW

    def _step(j, c):
        off = ebase + j * CH
        pltpu.sync_copy(gidx.at[pl.ds(off, CH)], idx_v.at[0])
        pltpu.sync_copy(sidx.at[pl.ds(off, CH)], idx_v.at[1])
        pltpu.async_copy(tbl_hbm.at[idx_v.at[0]], rows_v, sem).wait()
        pltpu.sync_copy(rows_v, acc_sh.at[idx_v.at[1]], add=True)
        return c

    lax.fori_loop(0, CW, _step, 0)
    plsc.subcore_barrier()

    out_r0 = cid * NP + sid * RPT
    pltpu.sync_copy(acc_sh.at[pl.ds(r0, RPT)], acc_out.at[pl.ds(out_r0, RPT)])


def _make_sc_kernel():
    mesh = plsc.VectorSubcoreMesh(core_axis_name="c", subcore_axis_name="s")
    out_type = jax.ShapeDtypeStruct((NC * NP, D), jnp.float32)
    scratch = [
        pltpu.VMEM((2, CH), jnp.int32),
        pltpu.VMEM((CH, D), jnp.float32),
        pltpu.VMEM_SHARED((NP, D), jnp.float32),
        pltpu.SemaphoreType.DMA,
    ]
    return pl.kernel(_sc_body, out_type=out_type, mesh=mesh,
                     scratch_types=scratch)


_sc_scatter = _make_sc_kernel()

_B = 2000


def _dense_body(off, h_ref, ax_ref, rc_ref, Wm_ref, bm_ref, Ws_ref,
                bs_ref, Wn_ref, bn_ref, o_ref):
    aggx = ax_ref[0] + ax_ref[1]
    rels = rc_ref[0] + rc_ref[1]
    deg = rels[:, 2 * RD:2 * RD + 1]
    hr = rels[:, off:off + RD]
    Wm = Wm_ref[...]
    num = (jnp.dot(aggx, Wm[:D], precision=_HI, preferred_element_type=jnp.float32)
           + jnp.dot(hr, Wm[D:], precision=_HI, preferred_element_type=jnp.float32)
           + deg * bm_ref[...])
    agg = num / jnp.maximum(deg, 1.0)
    out = (jnp.dot(h_ref[...], Ws_ref[...], precision=_HI, preferred_element_type=jnp.float32)
           + bs_ref[...]
           + jnp.dot(agg, Wn_ref[...], precision=_HI, preferred_element_type=jnp.float32)
           + bn_ref[...])
    o_ref[...] = jnp.maximum(out, 0.0)


def _dense_layer(off, h, aggx2, relacc2, Wm, bm, Ws, bs, Wn, bn):
    grid = (N // _B,)
    full = lambda i: (0, 0)
    return pl.pallas_call(
        functools.partial(_dense_body, off),
        grid=grid,
        in_specs=[
            pl.BlockSpec((_B, D), lambda i: (i, 0)),
            pl.BlockSpec((NC, _B, D), lambda i: (0, i, 0)),
            pl.BlockSpec((NC, _B, D), lambda i: (0, i, 0)),
            pl.BlockSpec((D + RD, D), full),
            pl.BlockSpec((1, D), full),
            pl.BlockSpec((D, D), full),
            pl.BlockSpec((1, D), full),
            pl.BlockSpec((D, D), full),
            pl.BlockSpec((1, D), full),
        ],
        out_specs=pl.BlockSpec((_B, D), lambda i: (i, 0)),
        out_shape=jax.ShapeDtypeStruct((N, D), jnp.float32),
    )(h, aggx2, relacc2, Wm, bm.reshape(1, D), Ws, bs.reshape(1, D),
      Wn, bn.reshape(1, D))


def kernel(x, edge_src, edge_dst, rel_ids,
           rel_emb_0, Wm_0, bm_0, Ws_0, bs_0, Wn_0, bn_0,
           rel_emb_1, Wm_1, bm_1, Ws_1, bs_1, Wn_1, bn_1):
    pad = EP - E
    srcp = jnp.concatenate([edge_src, jnp.zeros((pad,), jnp.int32)])
    dstp = jnp.concatenate([edge_dst, jnp.full((pad,), N, jnp.int32)])
    relp = jnp.concatenate([rel_ids, jnp.zeros((pad,), jnp.int32)])

    reltab = (jnp.zeros((R, D), jnp.float32)
              .at[:, :RD].set(rel_emb_0)
              .at[:, RD:2 * RD].set(rel_emb_1)
              .at[:, 2 * RD].set(1.0))

    aggx0 = _sc_scatter(x, srcp, dstp).reshape(NC, NP, D)
    relacc = _sc_scatter(reltab, relp, dstp).reshape(NC, NP, D)
    h1 = _dense_layer(0, x, aggx0, relacc, Wm_0, bm_0, Ws_0, bs_0, Wn_0, bn_0)

    aggx1 = _sc_scatter(h1, srcp, dstp).reshape(NC, NP, D)
    h2 = _dense_layer(RD, h1, aggx1, relacc, Wm_1, bm_1, Ws_1, bs_1, Wn_1, bn_1)
    return h2


# speedup vs baseline: 3.7610x; 1.2831x over previous
import functools

import jax
import jax.numpy as jnp
from jax import lax
from jax.experimental import pallas as pl
from jax.experimental.pallas import tpu as pltpu
from jax.experimental.pallas import tpu_sc as plsc

N = 10000
D = 128
R = 16
RD = 16
E = 320000

NC = 2
NS = 16
NW = NC * NS
CH = 256
RK = 64
CW = -(-E // (NW * CH))
EW = CW * CH
EP = EW * NW
NP = -(-N // 128) * 128
RPT = NP // NS

_HI = jax.lax.Precision.HIGHEST


def _sc_body(tbl_hbm, gidx, sidx, acc_out, gidx_v, sidx_v, rows_v, acc_sh, sem):
    cid = lax.axis_index("c")
    sid = lax.axis_index("s")
    wid = sid * NC + cid
    zero16 = jnp.zeros((16,), jnp.float32)

    def _zrow(i, c):
        for j in range(D // 16):
            rows_v[i, pl.ds(j * 16, 16)] = zero16
        return c

    lax.fori_loop(0, CH, _zrow, 0)
    r0 = sid * RPT
    for k in range(RPT // CH):
        pltpu.sync_copy(rows_v, acc_sh.at[pl.ds(r0 + k * CH, CH)])
    rem = RPT % CH
    if rem:
        tail = r0 + (RPT // CH) * CH
        pltpu.sync_copy(rows_v.at[pl.ds(0, rem)], acc_sh.at[pl.ds(tail, rem)])
    plsc.subcore_barrier()

    ebase = wid * EW

    def _step(j, c):
        off = ebase + j * CH
        pltpu.sync_copy(gidx.at[pl.ds(off, CH)], gidx_v)
        pltpu.sync_copy(sidx.at[pl.ds(off, CH)], sidx_v)
        pltpu.async_copy(tbl_hbm.at[gidx_v], rows_v, sem).wait()
        pltpu.sync_copy(rows_v, acc_sh.at[sidx_v], add=True)
        return c

    lax.fori_loop(0, CW, _step, 0)
    plsc.subcore_barrier()

    out_r0 = cid * NP + sid * RPT
    pltpu.sync_copy(acc_sh.at[pl.ds(r0, RPT)], acc_out.at[pl.ds(out_r0, RPT)])


def _make_sc_kernel():
    mesh = plsc.VectorSubcoreMesh(core_axis_name="c", subcore_axis_name="s")
    out_type = jax.ShapeDtypeStruct((NC * NP, D), jnp.float32)
    scratch = [
        pltpu.VMEM((CH,), jnp.int32),
        pltpu.VMEM((CH,), jnp.int32),
        pltpu.VMEM((CH, D), jnp.float32),
        pltpu.VMEM_SHARED((NP, D), jnp.float32),
        pltpu.SemaphoreType.DMA,
    ]
    return pl.kernel(_sc_body, out_type=out_type, mesh=mesh,
                     scratch_types=scratch)


_sc_scatter = _make_sc_kernel()

_B = 2000


def _dense_body(off, h_ref, ax_ref, rc_ref, Wm_ref, bm_ref, Ws_ref,
                bs_ref, Wn_ref, bn_ref, o_ref):
    aggx = ax_ref[0] + ax_ref[1]
    rels = rc_ref[0] + rc_ref[1]
    deg = rels[:, 2 * RD:2 * RD + 1]
    hr = rels[:, off:off + RD]
    Wm = Wm_ref[...]
    num = (jnp.dot(aggx, Wm[:D], precision=_HI, preferred_element_type=jnp.float32)
           + jnp.dot(hr, Wm[D:], precision=_HI, preferred_element_type=jnp.float32)
           + deg * bm_ref[...])
    agg = num / jnp.maximum(deg, 1.0)
    out = (jnp.dot(h_ref[...], Ws_ref[...], precision=_HI, preferred_element_type=jnp.float32)
           + bs_ref[...]
           + jnp.dot(agg, Wn_ref[...], precision=_HI, preferred_element_type=jnp.float32)
           + bn_ref[...])
    o_ref[...] = jnp.maximum(out, 0.0)


def _dense_layer(off, h, aggx2, relacc2, Wm, bm, Ws, bs, Wn, bn):
    grid = (N // _B,)
    full = lambda i: (0, 0)
    return pl.pallas_call(
        functools.partial(_dense_body, off),
        grid=grid,
        in_specs=[
            pl.BlockSpec((_B, D), lambda i: (i, 0)),
            pl.BlockSpec((NC, _B, D), lambda i: (0, i, 0)),
            pl.BlockSpec((NC, _B, D), lambda i: (0, i, 0)),
            pl.BlockSpec((D + RD, D), full),
            pl.BlockSpec((1, D), full),
            pl.BlockSpec((D, D), full),
            pl.BlockSpec((1, D), full),
            pl.BlockSpec((D, D), full),
            pl.BlockSpec((1, D), full),
        ],
        out_specs=pl.BlockSpec((_B, D), lambda i: (i, 0)),
        out_shape=jax.ShapeDtypeStruct((N, D), jnp.float32),
    )(h, aggx2, relacc2, Wm, bm.reshape(1, D), Ws, bs.reshape(1, D),
      Wn, bn.reshape(1, D))


def _reltab_body(re0_ref, re1_ref, o_ref):
    re0 = re0_ref[...]
    re1 = re1_ref[...]
    ones = jnp.ones((R, 1), jnp.float32)
    zpad = jnp.zeros((R, D - 2 * RD - 1), jnp.float32)
    row = jnp.concatenate([re0, re1, ones, zpad], axis=1)
    o_ref[...] = jnp.broadcast_to(row[None], (RK, R, D)).reshape(RK * R, D)


def _build_reltab(rel_emb_0, rel_emb_1):
    return pl.pallas_call(
        _reltab_body,
        out_shape=jax.ShapeDtypeStruct((RK * R, D), jnp.float32),
    )(rel_emb_0, rel_emb_1)


def kernel(x, edge_src, edge_dst, rel_ids,
           rel_emb_0, Wm_0, bm_0, Ws_0, bs_0, Wn_0, bn_0,
           rel_emb_1, Wm_1, bm_1, Ws_1, bs_1, Wn_1, bn_1):
    pad = EP - E
    srcp = jnp.concatenate([edge_src, jnp.zeros((pad,), jnp.int32)])
    dstp = jnp.concatenate([edge_dst, jnp.full((pad,), N, jnp.int32)])
    relp = jnp.concatenate([rel_ids, jnp.zeros((pad,), jnp.int32)])

    reltab = _build_reltab(rel_emb_0, rel_emb_1)
    relp = relp + R * (jnp.arange(EP, dtype=jnp.int32) % RK)

    aggx0 = _sc_scatter(x, srcp, dstp).reshape(NC, NP, D)
    relacc = _sc_scatter(reltab, relp, dstp).reshape(NC, NP, D)
    h1 = _dense_layer(0, x, aggx0, relacc, Wm_0, bm_0, Ws_0, bs_0, Wn_0, bn_0)

    aggx1 = _sc_scatter(h1, srcp, dstp).reshape(NC, NP, D)
    h2 = _dense_layer(RD, h1, aggx1, relacc, Wm_1, bm_1, Ws_1, bs_1, Wn_1, bn_1)
    return h2


# speedup vs baseline: 3.7617x; 1.0002x over previous
import functools

import jax
import jax.numpy as jnp
from jax import lax
from jax.experimental import pallas as pl
from jax.experimental.pallas import tpu as pltpu
from jax.experimental.pallas import tpu_sc as plsc

N = 10000
D = 128
R = 16
RD = 16
E = 320000

NC = 2
NS = 16
NW = NC * NS
CH = 256
RK = 64
CW = -(-E // (NW * CH))
EW = CW * CH
EP = EW * NW
NP = -(-N // 128) * 128
RPT = NP // NS

_HI = jax.lax.Precision.HIGHEST


def _sc_body(tbl_hbm, gidx, sidx, acc_out, gidx_v, sidx_v, rows_v,
             acc_sh, sem):
    cid = lax.axis_index("c")
    sid = lax.axis_index("s")
    wid = sid * NC + cid
    ebase = wid * EW
    zero16 = jnp.zeros((16,), jnp.float32)

    def _zrow(i, c):
        for j in range(D // 16):
            rows_v[i, pl.ds(j * 16, 16)] = zero16
        return c

    lax.fori_loop(0, CH, _zrow, 0)
    r0 = sid * RPT
    for k in range(RPT // CH):
        pltpu.sync_copy(rows_v, acc_sh.at[pl.ds(r0 + k * CH, CH)])
    rem = RPT % CH
    if rem:
        tail = r0 + (RPT // CH) * CH
        pltpu.sync_copy(rows_v.at[pl.ds(0, rem)], acc_sh.at[pl.ds(tail, rem)])
    plsc.subcore_barrier()

    def _chunk(j, c):
        e0 = ebase + j * CH
        pltpu.sync_copy(gidx.at[pl.ds(e0, CH)], gidx_v)
        pltpu.sync_copy(sidx.at[pl.ds(e0, CH)], sidx_v)
        pltpu.async_copy(tbl_hbm.at[gidx_v], rows_v, sem)
        pltpu.make_async_copy(tbl_hbm.at[pl.ds(0, CH)], rows_v, sem).wait()
        pltpu.sync_copy(rows_v, acc_sh.at[sidx_v], add=True)
        return c

    lax.fori_loop(0, CW, _chunk, 0)
    plsc.subcore_barrier()

    out_r0 = cid * NP + sid * RPT
    pltpu.sync_copy(acc_sh.at[pl.ds(r0, RPT)], acc_out.at[pl.ds(out_r0, RPT)])


def _make_sc_kernel():
    mesh = plsc.VectorSubcoreMesh(core_axis_name="c", subcore_axis_name="s")
    out_type = jax.ShapeDtypeStruct((NC * NP, D), jnp.float32)
    scratch = [
        pltpu.VMEM((CH,), jnp.int32),
        pltpu.VMEM((CH,), jnp.int32),
        pltpu.VMEM((CH, D), jnp.float32),
        pltpu.VMEM_SHARED((NP, D), jnp.float32),
        pltpu.SemaphoreType.DMA,
    ]
    return pl.kernel(_sc_body, out_type=out_type, mesh=mesh,
                     scratch_types=scratch)


_sc_scatter = _make_sc_kernel()

_B = 2000


def _dense_body(off, h_ref, ax_ref, rc_ref, Wm_ref, bm_ref, Ws_ref,
                bs_ref, Wn_ref, bn_ref, o_ref):
    aggx = ax_ref[0] + ax_ref[1]
    rels = rc_ref[0] + rc_ref[1]
    deg = rels[:, 2 * RD:2 * RD + 1]
    hr = rels[:, off:off + RD]
    Wm = Wm_ref[...]
    num = (jnp.dot(aggx, Wm[:D], precision=_HI, preferred_element_type=jnp.float32)
           + jnp.dot(hr, Wm[D:], precision=_HI, preferred_element_type=jnp.float32)
           + deg * bm_ref[...])
    agg = num / jnp.maximum(deg, 1.0)
    out = (jnp.dot(h_ref[...], Ws_ref[...], precision=_HI, preferred_element_type=jnp.float32)
           + bs_ref[...]
           + jnp.dot(agg, Wn_ref[...], precision=_HI, preferred_element_type=jnp.float32)
           + bn_ref[...])
    o_ref[...] = jnp.maximum(out, 0.0)


def _dense_layer(off, h, aggx2, relacc2, Wm, bm, Ws, bs, Wn, bn):
    grid = (N // _B,)
    full = lambda i: (0, 0)
    return pl.pallas_call(
        functools.partial(_dense_body, off),
        grid=grid,
        in_specs=[
            pl.BlockSpec((_B, D), lambda i: (i, 0)),
            pl.BlockSpec((NC, _B, D), lambda i: (0, i, 0)),
            pl.BlockSpec((NC, _B, D), lambda i: (0, i, 0)),
            pl.BlockSpec((D + RD, D), full),
            pl.BlockSpec((1, D), full),
            pl.BlockSpec((D, D), full),
            pl.BlockSpec((1, D), full),
            pl.BlockSpec((D, D), full),
            pl.BlockSpec((1, D), full),
        ],
        out_specs=pl.BlockSpec((_B, D), lambda i: (i, 0)),
        out_shape=jax.ShapeDtypeStruct((N, D), jnp.float32),
    )(h, aggx2, relacc2, Wm, bm.reshape(1, D), Ws, bs.reshape(1, D),
      Wn, bn.reshape(1, D))


def _reltab_body(re0_ref, re1_ref, o_ref):
    re0 = re0_ref[...]
    re1 = re1_ref[...]
    ones = jnp.ones((R, 1), jnp.float32)
    zpad = jnp.zeros((R, D - 2 * RD - 1), jnp.float32)
    row = jnp.concatenate([re0, re1, ones, zpad], axis=1)
    o_ref[...] = jnp.broadcast_to(row[None], (RK, R, D)).reshape(RK * R, D)


def _build_reltab(rel_emb_0, rel_emb_1):
    return pl.pallas_call(
        _reltab_body,
        out_shape=jax.ShapeDtypeStruct((RK * R, D), jnp.float32),
    )(rel_emb_0, rel_emb_1)


def kernel(x, edge_src, edge_dst, rel_ids,
           rel_emb_0, Wm_0, bm_0, Ws_0, bs_0, Wn_0, bn_0,
           rel_emb_1, Wm_1, bm_1, Ws_1, bs_1, Wn_1, bn_1):
    pad = EP - E
    srcp = jnp.concatenate([edge_src, jnp.zeros((pad,), jnp.int32)])
    dstp = jnp.concatenate([edge_dst, jnp.full((pad,), N, jnp.int32)])
    relp = jnp.concatenate([rel_ids, jnp.zeros((pad,), jnp.int32)])

    reltab = _build_reltab(rel_emb_0, rel_emb_1)
    relp = relp + R * (jnp.arange(EP, dtype=jnp.int32) % RK)

    aggx0 = _sc_scatter(x, srcp, dstp).reshape(NC, NP, D)
    relacc = _sc_scatter(reltab, relp, dstp).reshape(NC, NP, D)
    h1 = _dense_layer(0, x, aggx0, relacc, Wm_0, bm_0, Ws_0, bs_0, Wn_0, bn_0)

    aggx1 = _sc_scatter(h1, srcp, dstp).reshape(NC, NP, D)
    h2 = _dense_layer(RD, h1, aggx1, relacc, Wm_1, bm_1, Ws_1, bs_1, Wn_1, bn_1)
    return h2


# speedup vs baseline: 3.8569x; 1.0253x over previous
import functools

import jax
import jax.numpy as jnp
from jax import lax
from jax.experimental import pallas as pl
from jax.experimental.pallas import tpu as pltpu
from jax.experimental.pallas import tpu_sc as plsc

N = 10000
D = 128
R = 16
RD = 16
E = 320000

NC = 2
NS = 16
NW = NC * NS
CH = 192
RK = 64
CW = -(-(-(-E // (NW * CH))) // 2) * 2
NG = CW // 2
EW = CW * CH
EP = EW * NW
NP = -(-N // 128) * 128
RPT = NP // NS

_HI = jax.lax.Precision.HIGHEST


def _sc_body(tbl_hbm, gidx, sidx, acc_out, g0_v, s0_v, g1_v, s1_v,
             rows0_v, rows1_v, acc_sh, sem0, sem1):
    cid = lax.axis_index("c")
    sid = lax.axis_index("s")
    wid = sid * NC + cid
    ebase = wid * EW
    bufs = [(g0_v, s0_v, rows0_v, sem0), (g1_v, s1_v, rows1_v, sem1)]
    zero16 = jnp.zeros((16,), jnp.float32)

    def _zrow(i, c):
        for j in range(D // 16):
            rows0_v[i, pl.ds(j * 16, 16)] = zero16
        return c

    lax.fori_loop(0, CH, _zrow, 0)
    r0 = sid * RPT
    for k in range(RPT // CH):
        pltpu.sync_copy(rows0_v, acc_sh.at[pl.ds(r0 + k * CH, CH)])
    rem = RPT % CH
    if rem:
        tail = r0 + (RPT // CH) * CH
        pltpu.sync_copy(rows0_v.at[pl.ds(0, rem)], acc_sh.at[pl.ds(tail, rem)])
    plsc.subcore_barrier()

    def _load_start(j, gv, sv, rv, sem):
        e0 = ebase + j * CH
        pltpu.sync_copy(gidx.at[pl.ds(e0, CH)], gv)
        pltpu.sync_copy(sidx.at[pl.ds(e0, CH)], sv)
        pltpu.async_copy(tbl_hbm.at[gv], rv, sem)

    def _finish(sv, rv, sem):
        pltpu.make_async_copy(tbl_hbm.at[pl.ds(0, CH)], rv, sem).wait()
        pltpu.sync_copy(rv, acc_sh.at[sv], add=True)

    for b in range(2):
        _load_start(b, *bufs[b])

    def _grp(g, c):
        for b in range(2):
            gv, sv, rv, sem = bufs[b]
            _finish(sv, rv, sem)
            _load_start(g * 2 + b + 2, gv, sv, rv, sem)
        return c

    lax.fori_loop(0, NG - 1, _grp, 0)
    for b in range(2):
        _, sv, rv, sem = bufs[b]
        _finish(sv, rv, sem)
    plsc.subcore_barrier()

    out_r0 = cid * NP + sid * RPT
    pltpu.sync_copy(acc_sh.at[pl.ds(r0, RPT)], acc_out.at[pl.ds(out_r0, RPT)])


def _make_sc_kernel():
    mesh = plsc.VectorSubcoreMesh(core_axis_name="c", subcore_axis_name="s")
    out_type = jax.ShapeDtypeStruct((NC * NP, D), jnp.float32)
    scratch = [
        pltpu.VMEM((CH,), jnp.int32),
        pltpu.VMEM((CH,), jnp.int32),
        pltpu.VMEM((CH,), jnp.int32),
        pltpu.VMEM((CH,), jnp.int32),
        pltpu.VMEM((CH, D), jnp.float32),
        pltpu.VMEM((CH, D), jnp.float32),
        pltpu.VMEM_SHARED((NP, D), jnp.float32),
        pltpu.SemaphoreType.DMA,
        pltpu.SemaphoreType.DMA,
    ]
    return pl.kernel(_sc_body, out_type=out_type, mesh=mesh,
                     scratch_types=scratch)


_sc_scatter = _make_sc_kernel()

_B = 2000


def _dense_body(off, h_ref, ax_ref, rc_ref, Wm_ref, bm_ref, Ws_ref,
                bs_ref, Wn_ref, bn_ref, o_ref):
    aggx = ax_ref[0] + ax_ref[1]
    rels = rc_ref[0] + rc_ref[1]
    deg = rels[:, 2 * RD:2 * RD + 1]
    hr = rels[:, off:off + RD]
    Wm = Wm_ref[...]
    num = (jnp.dot(aggx, Wm[:D], precision=_HI, preferred_element_type=jnp.float32)
           + jnp.dot(hr, Wm[D:], precision=_HI, preferred_element_type=jnp.float32)
           + deg * bm_ref[...])
    agg = num / jnp.maximum(deg, 1.0)
    out = (jnp.dot(h_ref[...], Ws_ref[...], precision=_HI, preferred_element_type=jnp.float32)
           + bs_ref[...]
           + jnp.dot(agg, Wn_ref[...], precision=_HI, preferred_element_type=jnp.float32)
           + bn_ref[...])
    o_ref[...] = jnp.maximum(out, 0.0)


def _dense_layer(off, h, aggx2, relacc2, Wm, bm, Ws, bs, Wn, bn):
    grid = (N // _B,)
    full = lambda i: (0, 0)
    return pl.pallas_call(
        functools.partial(_dense_body, off),
        grid=grid,
        in_specs=[
            pl.BlockSpec((_B, D), lambda i: (i, 0)),
            pl.BlockSpec((NC, _B, D), lambda i: (0, i, 0)),
            pl.BlockSpec((NC, _B, D), lambda i: (0, i, 0)),
            pl.BlockSpec((D + RD, D), full),
            pl.BlockSpec((1, D), full),
            pl.BlockSpec((D, D), full),
            pl.BlockSpec((1, D), full),
            pl.BlockSpec((D, D), full),
            pl.BlockSpec((1, D), full),
        ],
        out_specs=pl.BlockSpec((_B, D), lambda i: (i, 0)),
        out_shape=jax.ShapeDtypeStruct((N, D), jnp.float32),
    )(h, aggx2, relacc2, Wm, bm.reshape(1, D), Ws, bs.reshape(1, D),
      Wn, bn.reshape(1, D))


def _reltab_body(re0_ref, re1_ref, o_ref):
    re0 = re0_ref[...]
    re1 = re1_ref[...]
    ones = jnp.ones((R, 1), jnp.float32)
    zpad = jnp.zeros((R, D - 2 * RD - 1), jnp.float32)
    row = jnp.concatenate([re0, re1, ones, zpad], axis=1)
    o_ref[...] = jnp.broadcast_to(row[None], (RK, R, D)).reshape(RK * R, D)


def _build_reltab(rel_emb_0, rel_emb_1):
    return pl.pallas_call(
        _reltab_body,
        out_shape=jax.ShapeDtypeStruct((RK * R, D), jnp.float32),
    )(rel_emb_0, rel_emb_1)


def kernel(x, edge_src, edge_dst, rel_ids,
           rel_emb_0, Wm_0, bm_0, Ws_0, bs_0, Wn_0, bn_0,
           rel_emb_1, Wm_1, bm_1, Ws_1, bs_1, Wn_1, bn_1):
    pad = EP - E
    srcp = jnp.concatenate([edge_src, jnp.zeros((pad,), jnp.int32)])
    dstp = jnp.concatenate([edge_dst, jnp.full((pad,), N, jnp.int32)])
    relp = jnp.concatenate([rel_ids, jnp.zeros((pad,), jnp.int32)])

    reltab = _build_reltab(rel_emb_0, rel_emb_1)
    relp = relp + R * (jnp.arange(EP, dtype=jnp.int32) % RK)

    aggx0 = _sc_scatter(x, srcp, dstp).reshape(NC, NP, D)
    relacc = _sc_scatter(reltab, relp, dstp).reshape(NC, NP, D)
    h1 = _dense_layer(0, x, aggx0, relacc, Wm_0, bm_0, Ws_0, bs_0, Wn_0, bn_0)

    aggx1 = _sc_scatter(h1, srcp, dstp).reshape(NC, NP, D)
    h2 = _dense_layer(RD, h1, aggx1, relacc, Wm_1, bm_1, Ws_1, bs_1, Wn_1, bn_1)
    return h2
